# Initial kernel scaffold; baseline (speedup 1.0000x reference)
import jax
import jax.numpy as jnp
from jax.experimental import pallas as pl


def kernel(V, F):
    raise NotImplementedError("write your pallas kernel here")


# speedup vs baseline: 113.1741x; 113.1741x over previous
import jax
import jax.numpy as jnp
from jax import lax
from jax.experimental import pallas as pl
from jax.experimental.pallas import tpu as pltpu
from jax.experimental.pallas import tpu_sc as plsc

B, N, FC = 4, 100000, 200000
BN = B * N
NF = B * FC
NC, NS, L = 2, 16, 16
NW = NC * NS
CH = 1600
NCH = NF // CH
CPW = -(-NCH // NW)
INNER = CH // L
ZROWS = 5000
NZ = BN // NS // ZROWS
SPT = BN // NS


def _rsqrt(q):
    yi = jnp.int32(0x5F3759DF) - lax.shift_right_arithmetic(
        lax.bitcast_convert_type(q, jnp.int32), 1)
    y = lax.bitcast_convert_type(yi, jnp.float32)
    h = q * 0.5
    y = y * (1.5 - h * y * y)
    y = y * (1.5 - h * y * y)
    y = y * (1.5 - h * y * y)
    return y


def _sc_body(vx, vy, vz, f0_hbm, f1_hbm, f2_hbm, z_hbm, out_hbm,
             accx, accy, accz,
             idx0, idx1, idx2,
             p00, p01, p02, p10, p11, p12, p20, p21, p22,
             g00, g01, g02, g10, g11, g12, g20, g21, g22,
             tmp):
    c = lax.axis_index("c")
    s = lax.axis_index("s")
    w = c * NS + s
    acc = (accx, accy, accz)
    v_hbm = (vx, vy, vz)
    idx = (idx0, idx1, idx2)
    p = ((p00, p01, p02), (p10, p11, p12), (p20, p21, p22))
    g = ((g00, g01, g02), (g10, g11, g12), (g20, g21, g22))

    pltpu.sync_copy(z_hbm, tmp)
    row0 = s * SPT
    for ax in range(3):
        for j in range(NZ):
            pltpu.sync_copy(tmp, acc[ax].at[pl.ds(row0 + j * ZROWS, ZROWS)])

    plsc.subcore_barrier()

    def _chunk(k, _):
        cid = k * NW + w

        @pl.when(cid < NCH)
        def _():
            base = cid * CH
            for v in range(3):
                pltpu.sync_copy((f0_hbm, f1_hbm, f2_hbm)[v].at[pl.ds(base, CH)],
                                idx[v])
            for v in range(3):
                for ax in range(3):
                    pltpu.sync_copy(v_hbm[ax].at[idx[v]], p[v][ax])

            def _faces(i, _):
                o = i * L
                p0 = [p[0][ax][pl.ds(o, L)] for ax in range(3)]
                p1 = [p[1][ax][pl.ds(o, L)] for ax in range(3)]
                p2 = [p[2][ax][pl.ds(o, L)] for ax in range(3)]
                a = sum((p1[j] - p2[j]) * (p1[j] - p2[j]) for j in range(3))
                b = sum((p2[j] - p0[j]) * (p2[j] - p0[j]) for j in range(3))
                cc = sum((p0[j] - p1[j]) * (p0[j] - p1[j]) for j in range(3))
                t = a + b + cc
                q = t * t - 2.0 * (a * a + b * b + cc * cc)
                r2 = 0.5 * _rsqrt(q)
                c0 = (t - a - a) * r2
                c1 = (t - b - b) * r2
                c2 = (t - cc - cc) * r2
                s0 = c1 + c2
                s1 = c0 + c2
                s2 = c0 + c1
                for j in range(3):
                    g[0][j][pl.ds(o, L)] = c2 * p1[j] + c1 * p2[j] - s0 * p0[j]
                    g[1][j][pl.ds(o, L)] = c0 * p2[j] + c2 * p0[j] - s1 * p1[j]
                    g[2][j][pl.ds(o, L)] = c1 * p0[j] + c0 * p1[j] - s2 * p2[j]
                return 0
            lax.fori_loop(0, INNER, _faces, 0)

            for v in range(3):
                for ax in range(3):
                    pltpu.sync_copy(g[v][ax], acc[ax].at[idx[v]], add=True)
        return 0
    lax.fori_loop(0, CPW, _chunk, 0)

    plsc.subcore_barrier()

    for ax in range(3):
        for j in range(NZ):
            r = row0 + j * ZROWS
            pltpu.sync_copy(acc[ax].at[pl.ds(r, ZROWS)], tmp)
            pltpu.sync_copy(tmp, out_hbm.at[pl.ds((c * 3 + ax) * BN + r, ZROWS)])


_sc_call = pl.kernel(
    _sc_body,
    out_type=jax.ShapeDtypeStruct((NC * 3 * BN,), jnp.float32),
    mesh=plsc.VectorSubcoreMesh(core_axis_name="c", subcore_axis_name="s"),
    scratch_types=(
        [pltpu.VMEM_SHARED((BN,), jnp.float32)] * 3
        + [pltpu.VMEM((CH,), jnp.int32)] * 3
        + [pltpu.VMEM((CH,), jnp.float32)] * 18
        + [pltpu.VMEM((ZROWS,), jnp.float32)]
    ),
)


def _tc_add_body(a_ref, o_ref):
    o_ref[...] = a_ref[0] + a_ref[1]


def kernel(V, F):
    Vf = V.reshape(BN, 3)
    Fi = F.astype(jnp.int32)
    off = (jnp.arange(B, dtype=jnp.int32) * N)[:, None]
    f0 = (Fi[:, :, 0] + off).reshape(-1)
    f1 = (Fi[:, :, 1] + off).reshape(-1)
    f2 = (Fi[:, :, 2] + off).reshape(-1)
    z = jnp.zeros((ZROWS,), jnp.float32)
    parts = _sc_call(Vf[:, 0], Vf[:, 1], Vf[:, 2], f0, f1, f2, z)
    summed = pl.pallas_call(
        _tc_add_body,
        out_shape=jax.ShapeDtypeStruct((3 * BN // 128, 128), jnp.float32),
    )(parts.reshape(NC, 3 * BN // 128, 128))
    return summed.reshape(3, BN).T
